# baseline (speedup 1.0000x reference)
import functools

import jax
import jax.numpy as jnp
from jax.experimental import pallas as pl
from jax.experimental.pallas import tpu as pltpu
from jax.experimental.pallas import tpu_sc as plsc

N = 10000
C = 128
K = 20
R = 400
NT = N // R
CT = 1024
NCT = 10
NPAD = 10240
W = 32
GWIN = 256
F32MAX = float(jnp.finfo(jnp.float32).max)
I32BIG = int(jnp.iinfo(jnp.int32).max)

CHUNKS = (2400, 2400, 2400, 2800)


def _sqnorm_body(x_ref, sq_ref):
    xr = x_ref[...]
    sq_ref[...] = jnp.sum(xr * xr, axis=1, keepdims=True)


def _sqnorm(x, interpret=False):
    return pl.pallas_call(
        _sqnorm_body,
        grid=(NT,),
        in_specs=[pl.BlockSpec((R, C), lambda r: (r, 0))],
        out_specs=pl.BlockSpec((R, 1), lambda r: (r, 0)),
        out_shape=jax.ShapeDtypeStruct((N, 1), jnp.float32),
        interpret=interpret,
    )(x)


def _rot(x, s):
    return jnp.concatenate([x[:, s:], x[:, :s]], axis=1)


def _topk_body(
    base_t, jlo_ref, jhi_ref, xr_ref, xp_ref, sqr_ref, sqc_ref, rs_ref,
    re_ref, idx_ref
):
    t = pl.program_id(0) + base_t
    xr = xr_ref[...]
    sqr = sqr_ref[...]
    rs = rs_ref[...]
    re = re_ref[...]
    row_ids = t * R + jax.lax.broadcasted_iota(jnp.int32, (R, 1), 0)
    lane_w2 = jax.lax.broadcasted_iota(jnp.int32, (1, 2 * W), 1)

    v0 = jnp.full((R, W), I32BIG, dtype=jnp.int32)
    i0 = jnp.broadcast_to(row_ids, (R, W))

    def tile_body(j, carry):
        v, idxs = carry
        cs = j * CT
        xc = xp_ref[pl.ds(cs, CT), :]
        dots = jax.lax.dot_general(
            xr, xc, (((1,), (1,)), ((), ())),
            preferred_element_type=jnp.float32,
        )
        sqc = sqc_ref[j]
        lane_c = jax.lax.broadcasted_iota(jnp.int32, (1, CT), 1)
        colid = cs + lane_c
        valid = (colid >= rs) & (colid < re)
        s = ((sqr + sqc) - 2.0 * dots + 16.0) * 512.0
        s = jnp.clip(s, 0.0, 2090000.0)
        ki = s.astype(jnp.int32)
        kk = jnp.where(valid, (ki << 10) | lane_c, I32BIG)

        acc = jnp.concatenate(
            [v, jnp.full((R, W), I32BIG, dtype=jnp.int32)], axis=1
        )
        cac = jnp.concatenate([idxs, idxs], axis=1)
        for q in range(K):
            m = jnp.min(kk, axis=1, keepdims=True)
            tgt = lane_w2 == (2 * W - 1 - q)
            acc = jnp.where(tgt, m, acc)
            cac = jnp.where(tgt, cs + (m & (CT - 1)), cac)
            kk = jnp.where(kk == m, I32BIG, kk)
        st = W
        while st >= 1:
            pk = jnp.where((lane_w2 & st) == 0, _rot(acc, st), _rot(acc, 2 * W - st))
            pc = jnp.where((lane_w2 & st) == 0, _rot(cac, st), _rot(cac, 2 * W - st))
            lower = (lane_w2 & st) == 0
            take = (lower & (pk < acc)) | (~lower & (pk > acc))
            acc = jnp.where(take, pk, acc)
            cac = jnp.where(take, pc, cac)
            st //= 2
        return acc[:, :W], cac[:, :W]

    _, idxs = jax.lax.fori_loop(jlo_ref[t], jhi_ref[t], tile_body, (v0, i0))
    idx_ref[...] = jnp.minimum(idxs[:, :K], N - 1)


def _topk(xp_bf, sq, sq3, jlo, jhi, row_start, row_end, base_t, nrows,
          interpret=False):
    return pl.pallas_call(
        functools.partial(_topk_body, base_t),
        grid=(nrows // R,),
        in_specs=[
            pl.BlockSpec(memory_space=pltpu.SMEM),
            pl.BlockSpec(memory_space=pltpu.SMEM),
            pl.BlockSpec((R, C), lambda r: (r + base_t, 0)),
            pl.BlockSpec((NPAD, C), lambda r: (0, 0)),
            pl.BlockSpec((R, 1), lambda r: (r + base_t, 0)),
            pl.BlockSpec((NCT, 1, CT), lambda r: (0, 0, 0)),
            pl.BlockSpec((R, 1), lambda r: (r + base_t, 0)),
            pl.BlockSpec((R, 1), lambda r: (r + base_t, 0)),
        ],
        out_specs=pl.BlockSpec((R, K), lambda r: (r, 0)),
        out_shape=jax.ShapeDtypeStruct((nrows, K), jnp.int32),
        interpret=interpret,
    )(jlo, jhi, xp_bf[:N], xp_bf, sq, sq3, row_start, row_end)


def _gather_sc(x, idx_flat_padded, npad):
    mesh = plsc.VectorSubcoreMesh(core_axis_name="c", subcore_axis_name="s")

    @pl.kernel(
        out_type=jax.ShapeDtypeStruct((npad, C), jnp.float32),
        mesh=mesh,
    )
    def kern(x_hbm, i_hbm, o_hbm):
        def body(i_vmem, o_vmem):
            pltpu.sync_copy(x_hbm.at[i_vmem.at[0]], o_vmem)

        pltpu.emit_pipeline(
            body,
            grid=(npad // GWIN,),
            in_specs=[pl.BlockSpec((1, GWIN), index_map=lambda i: (0, i))],
            out_specs=[pl.BlockSpec((GWIN, C), index_map=lambda i: (i, 0))],
            core_axis_name=("c", "s"),
            dimension_semantics=(pltpu.PARALLEL,),
        )(i_hbm, o_hbm)

    return kern(x, idx_flat_padded)


def _final_body(x_ref, g_ref, w1_ref, b1_ref, w2_ref, b2_ref, o_ref):
    xi = x_ref[...]
    xi_bf = xi.astype(jnp.bfloat16)
    w1 = w1_ref[...].astype(jnp.bfloat16)
    w2 = w2_ref[...].astype(jnp.bfloat16)
    b1 = b1_ref[...]
    b2 = b2_ref[...]
    acc = None
    for k in range(K):
        dj = (g_ref[k] - xi).astype(jnp.bfloat16)
        e = jnp.concatenate([xi_bf, dj], axis=1)
        h1 = jnp.maximum(
            jnp.dot(e, w1, preferred_element_type=jnp.float32) + b1, 0.0
        )
        z = jnp.dot(
            h1.astype(jnp.bfloat16), w2, preferred_element_type=jnp.float32
        ) + b2
        h2 = jnp.maximum(z, 0.0)
        acc = h2 if acc is None else jnp.maximum(acc, h2)
    o_ref[...] = acc


def _final(x, G, W1, b1, W2, b2, base_t, nrows, interpret=False):
    return pl.pallas_call(
        _final_body,
        grid=(nrows // R,),
        in_specs=[
            pl.BlockSpec((R, C), lambda r: (r + base_t, 0)),
            pl.BlockSpec((K, R, C), lambda r: (0, r, 0)),
            pl.BlockSpec((2 * C, C), lambda r: (0, 0)),
            pl.BlockSpec((1, C), lambda r: (0, 0)),
            pl.BlockSpec((C, C), lambda r: (0, 0)),
            pl.BlockSpec((1, C), lambda r: (0, 0)),
        ],
        out_specs=pl.BlockSpec((R, C), lambda r: (r, 0)),
        out_shape=jax.ShapeDtypeStruct((nrows, C), jnp.float32),
        interpret=interpret,
    )(x, G, W1, b1.reshape(1, C), W2, b2.reshape(1, C))


def _segment_scalars(batch):
    gids = jnp.arange(8, dtype=batch.dtype)
    seg_start = jnp.searchsorted(batch, gids, side="left").astype(jnp.int32)
    seg_end = jnp.searchsorted(batch, gids, side="right").astype(jnp.int32)
    row_start = seg_start[batch].reshape(N, 1)
    row_end = seg_end[batch].reshape(N, 1)
    first_b = batch[::R]
    last_b = batch[R - 1 :: R]
    jlo = (seg_start[first_b] // CT).astype(jnp.int32)
    jhi = ((seg_end[last_b] + CT - 1) // CT).astype(jnp.int32)
    return jlo, jhi, row_start, row_end


def _gather_pad(idx_half):
    nk = idx_half.shape[0] * K
    npad = -(-nk // GWIN) * GWIN
    idx_km = jnp.transpose(idx_half).reshape(-1)
    return jnp.pad(idx_km, (0, npad - nk)).reshape(1, npad), npad


@jax.jit
def kernel(x, batch, W1, b1, W2, b2):
    batch = batch.astype(jnp.int32)
    xp_bf = jnp.pad(x.astype(jnp.bfloat16), ((0, NPAD - N), (0, 0)))
    jlo, jhi, row_start, row_end = _segment_scalars(batch)

    sq = _sqnorm(x)
    sq3 = jnp.pad(sq.reshape(N), (0, NPAD - N)).reshape(NCT, 1, CT)

    outs = []
    base = 0
    for nh in CHUNKS:
        bt = base // R
        idx = _topk(xp_bf, sq, sq3, jlo, jhi, row_start, row_end, bt, nh)
        isc, npd = _gather_pad(idx)
        G = _gather_sc(x, isc, npd)[: nh * K].reshape(K, nh, C)
        outs.append(_final(x, G, W1, b1, W2, b2, bt, nh))
        base += nh
    return jnp.concatenate(outs, axis=0)


# speedup vs baseline: 1.0534x; 1.0534x over previous
import functools

import jax
import jax.numpy as jnp
from jax.experimental import pallas as pl
from jax.experimental.pallas import tpu as pltpu
from jax.experimental.pallas import tpu_sc as plsc

N = 10000
C = 128
K = 20
R = 400
NT = N // R
CT = 1024
NCT = 10
NPAD = 10240
W = 32
GWIN = 256
F32MAX = float(jnp.finfo(jnp.float32).max)
I32BIG = int(jnp.iinfo(jnp.int32).max)

CHUNKS = (4800, 5200)


def _sqnorm_body(x_ref, sq_ref):
    xr = x_ref[...]
    sq_ref[...] = jnp.sum(xr * xr, axis=1, keepdims=True)


def _sqnorm(x, interpret=False):
    return pl.pallas_call(
        _sqnorm_body,
        grid=(NT,),
        in_specs=[pl.BlockSpec((R, C), lambda r: (r, 0))],
        out_specs=pl.BlockSpec((R, 1), lambda r: (r, 0)),
        out_shape=jax.ShapeDtypeStruct((N, 1), jnp.float32),
        interpret=interpret,
    )(x)


def _rot(x, s):
    return jnp.concatenate([x[:, s:], x[:, :s]], axis=1)


def _topk_body(
    base_t, jlo_ref, jhi_ref, xr_ref, xp_ref, sqr_ref, sqc_ref, rs_ref,
    re_ref, idx_ref
):
    t = pl.program_id(0) + base_t
    xr = xr_ref[...]
    sqr = sqr_ref[...]
    rs = rs_ref[...]
    re = re_ref[...]
    row_ids = t * R + jax.lax.broadcasted_iota(jnp.int32, (R, 1), 0)
    lane_w2 = jax.lax.broadcasted_iota(jnp.int32, (1, 2 * W), 1)

    v0 = jnp.full((R, W), I32BIG, dtype=jnp.int32)
    i0 = jnp.broadcast_to(row_ids, (R, W))

    def tile_body(j, carry):
        v, idxs = carry
        cs = j * CT
        xc = xp_ref[pl.ds(cs, CT), :]
        dots = jax.lax.dot_general(
            xr, xc, (((1,), (1,)), ((), ())),
            preferred_element_type=jnp.float32,
        )
        sqc = sqc_ref[j]
        lane_c = jax.lax.broadcasted_iota(jnp.int32, (1, CT), 1)
        colid = cs + lane_c
        valid = (colid >= rs) & (colid < re)
        s = ((sqr + sqc) - 2.0 * dots + 16.0) * 512.0
        s = jnp.clip(s, 0.0, 2090000.0)
        ki = s.astype(jnp.int32)
        kk = jnp.where(valid, (ki << 10) | lane_c, I32BIG)

        acc = jnp.concatenate(
            [v, jnp.full((R, W), I32BIG, dtype=jnp.int32)], axis=1
        )
        cac = jnp.concatenate([idxs, idxs], axis=1)
        for q in range(K):
            m = jnp.min(kk, axis=1, keepdims=True)
            tgt = lane_w2 == (2 * W - 1 - q)
            acc = jnp.where(tgt, m, acc)
            cac = jnp.where(tgt, cs + (m & (CT - 1)), cac)
            kk = jnp.where(kk == m, I32BIG, kk)
        st = W
        while st >= 1:
            pk = jnp.where((lane_w2 & st) == 0, _rot(acc, st), _rot(acc, 2 * W - st))
            pc = jnp.where((lane_w2 & st) == 0, _rot(cac, st), _rot(cac, 2 * W - st))
            lower = (lane_w2 & st) == 0
            take = (lower & (pk < acc)) | (~lower & (pk > acc))
            acc = jnp.where(take, pk, acc)
            cac = jnp.where(take, pc, cac)
            st //= 2
        return acc[:, :W], cac[:, :W]

    _, idxs = jax.lax.fori_loop(jlo_ref[t], jhi_ref[t], tile_body, (v0, i0))
    idx_ref[...] = jnp.minimum(idxs[:, :K], N - 1)


def _topk(xp_bf, sq, sq3, jlo, jhi, row_start, row_end, base_t, nrows,
          interpret=False):
    return pl.pallas_call(
        functools.partial(_topk_body, base_t),
        grid=(nrows // R,),
        in_specs=[
            pl.BlockSpec(memory_space=pltpu.SMEM),
            pl.BlockSpec(memory_space=pltpu.SMEM),
            pl.BlockSpec((R, C), lambda r: (r + base_t, 0)),
            pl.BlockSpec((NPAD, C), lambda r: (0, 0)),
            pl.BlockSpec((R, 1), lambda r: (r + base_t, 0)),
            pl.BlockSpec((NCT, 1, CT), lambda r: (0, 0, 0)),
            pl.BlockSpec((R, 1), lambda r: (r + base_t, 0)),
            pl.BlockSpec((R, 1), lambda r: (r + base_t, 0)),
        ],
        out_specs=pl.BlockSpec((R, K), lambda r: (r, 0)),
        out_shape=jax.ShapeDtypeStruct((nrows, K), jnp.int32),
        interpret=interpret,
    )(jlo, jhi, xp_bf[:N], xp_bf, sq, sq3, row_start, row_end)


def _gather_sc(x, idx_flat_padded, npad):
    mesh = plsc.VectorSubcoreMesh(core_axis_name="c", subcore_axis_name="s")

    @pl.kernel(
        out_type=jax.ShapeDtypeStruct((npad, C), jnp.float32),
        mesh=mesh,
    )
    def kern(x_hbm, i_hbm, o_hbm):
        def body(i_vmem, o_vmem):
            pltpu.sync_copy(x_hbm.at[i_vmem.at[0]], o_vmem)

        pltpu.emit_pipeline(
            body,
            grid=(npad // GWIN,),
            in_specs=[pl.BlockSpec((1, GWIN), index_map=lambda i: (0, i))],
            out_specs=[pl.BlockSpec((GWIN, C), index_map=lambda i: (i, 0))],
            core_axis_name=("c", "s"),
            dimension_semantics=(pltpu.PARALLEL,),
        )(i_hbm, o_hbm)

    return kern(x, idx_flat_padded)


def _final_body(x_ref, g_ref, w1_ref, b1_ref, w2_ref, b2_ref, o_ref):
    xi = x_ref[...]
    xi_bf = xi.astype(jnp.bfloat16)
    w1 = w1_ref[...].astype(jnp.bfloat16)
    w2 = w2_ref[...].astype(jnp.bfloat16)
    b1 = b1_ref[...]
    b2 = b2_ref[...]
    acc = None
    for k in range(K):
        dj = (g_ref[k] - xi).astype(jnp.bfloat16)
        e = jnp.concatenate([xi_bf, dj], axis=1)
        h1 = jnp.maximum(
            jnp.dot(e, w1, preferred_element_type=jnp.float32) + b1, 0.0
        )
        z = jnp.dot(
            h1.astype(jnp.bfloat16), w2, preferred_element_type=jnp.float32
        ) + b2
        h2 = jnp.maximum(z, 0.0)
        acc = h2 if acc is None else jnp.maximum(acc, h2)
    o_ref[...] = acc


def _final(x, G, W1, b1, W2, b2, base_t, nrows, interpret=False):
    return pl.pallas_call(
        _final_body,
        grid=(nrows // R,),
        in_specs=[
            pl.BlockSpec((R, C), lambda r: (r + base_t, 0)),
            pl.BlockSpec((K, R, C), lambda r: (0, r, 0)),
            pl.BlockSpec((2 * C, C), lambda r: (0, 0)),
            pl.BlockSpec((1, C), lambda r: (0, 0)),
            pl.BlockSpec((C, C), lambda r: (0, 0)),
            pl.BlockSpec((1, C), lambda r: (0, 0)),
        ],
        out_specs=pl.BlockSpec((R, C), lambda r: (r, 0)),
        out_shape=jax.ShapeDtypeStruct((nrows, C), jnp.float32),
        interpret=interpret,
    )(x, G, W1, b1.reshape(1, C), W2, b2.reshape(1, C))


def _segment_scalars(batch):
    gids = jnp.arange(8, dtype=batch.dtype)
    seg_start = jnp.searchsorted(batch, gids, side="left").astype(jnp.int32)
    seg_end = jnp.searchsorted(batch, gids, side="right").astype(jnp.int32)
    row_start = seg_start[batch].reshape(N, 1)
    row_end = seg_end[batch].reshape(N, 1)
    first_b = batch[::R]
    last_b = batch[R - 1 :: R]
    jlo = (seg_start[first_b] // CT).astype(jnp.int32)
    jhi = ((seg_end[last_b] + CT - 1) // CT).astype(jnp.int32)
    return jlo, jhi, row_start, row_end


def _gather_pad(idx_half):
    nk = idx_half.shape[0] * K
    npad = -(-nk // GWIN) * GWIN
    idx_km = jnp.transpose(idx_half).reshape(-1)
    return jnp.pad(idx_km, (0, npad - nk)).reshape(1, npad), npad


@jax.jit
def kernel(x, batch, W1, b1, W2, b2):
    batch = batch.astype(jnp.int32)
    xp_bf = jnp.pad(x.astype(jnp.bfloat16), ((0, NPAD - N), (0, 0)))
    jlo, jhi, row_start, row_end = _segment_scalars(batch)

    sq = _sqnorm(x)
    sq3 = jnp.pad(sq.reshape(N), (0, NPAD - N)).reshape(NCT, 1, CT)

    outs = []
    base = 0
    for nh in CHUNKS:
        bt = base // R
        idx = _topk(xp_bf, sq, sq3, jlo, jhi, row_start, row_end, bt, nh)
        isc, npd = _gather_pad(idx)
        G = _gather_sc(x, isc, npd)[: nh * K].reshape(K, nh, C)
        outs.append(_final(x, G, W1, b1, W2, b2, bt, nh))
        base += nh
    return jnp.concatenate(outs, axis=0)


# speedup vs baseline: 1.2482x; 1.1849x over previous
import functools

import jax
import jax.numpy as jnp
from jax.experimental import pallas as pl
from jax.experimental.pallas import tpu as pltpu
from jax.experimental.pallas import tpu_sc as plsc

N = 10000
C = 128
K = 20
R = 400
NT = N // R
CT = 1024
NCT = 10
NPAD = 10240
W = 32
GWIN = 256
F32MAX = float(jnp.finfo(jnp.float32).max)
I32BIG = int(jnp.iinfo(jnp.int32).max)

CHUNKS = (4800, 5200)


def _sqnorm_body(x_ref, sq_ref):
    xr = x_ref[...]
    sq_ref[...] = jnp.sum(xr * xr, axis=1, keepdims=True)


def _sqnorm(x, interpret=False):
    return pl.pallas_call(
        _sqnorm_body,
        grid=(NT,),
        in_specs=[pl.BlockSpec((R, C), lambda r: (r, 0))],
        out_specs=pl.BlockSpec((R, 1), lambda r: (r, 0)),
        out_shape=jax.ShapeDtypeStruct((N, 1), jnp.float32),
        interpret=interpret,
    )(x)


def _rot(x, s):
    return jnp.concatenate([x[:, s:], x[:, :s]], axis=1)


def _topk_body(
    base_t, jlo_ref, jhi_ref, xr_ref, xp_ref, sqr_ref, sqc_ref, rs_ref,
    re_ref, idx_ref
):
    t = pl.program_id(0) + base_t
    xr = xr_ref[...]
    sqr = sqr_ref[...]
    rs = rs_ref[...]
    re = re_ref[...]
    row_ids = t * R + jax.lax.broadcasted_iota(jnp.int32, (R, 1), 0)
    lane_w2 = jax.lax.broadcasted_iota(jnp.int32, (1, 2 * W), 1)

    v0 = jnp.full((R, W), F32MAX, dtype=jnp.float32)
    i0 = jnp.broadcast_to(row_ids, (R, W))

    def tile_body(j, carry):
        v, idxs = carry
        cs = j * CT
        xc = xp_ref[pl.ds(cs, CT), :]
        dots = jax.lax.dot_general(
            xr, xc, (((1,), (1,)), ((), ())),
            preferred_element_type=jnp.float32,
        )
        sqc = sqc_ref[j]
        lane_c = jax.lax.broadcasted_iota(jnp.int32, (1, CT), 1)
        colid = cs + lane_c
        valid = (colid >= rs) & (colid < re)
        s = ((sqr + sqc) - 2.0 * dots + 48.0) * 512.0
        s = jnp.clip(s, 16384.0, 2088000.0)
        ki = s.astype(jnp.int32)
        kf = jax.lax.bitcast_convert_type((ki << 10) | lane_c, jnp.float32)
        kk = jnp.where(valid, kf, F32MAX)

        acc = jnp.concatenate(
            [v, jnp.full((R, W), F32MAX, dtype=jnp.float32)], axis=1
        )
        cac = jnp.concatenate([idxs, idxs], axis=1)
        for q in range(K):
            m = jnp.min(kk, axis=1, keepdims=True)
            tgt = lane_w2 == (2 * W - 1 - q)
            acc = jnp.where(tgt, m, acc)
            mi = jax.lax.bitcast_convert_type(m, jnp.int32)
            cac = jnp.where(tgt, cs + (mi & (CT - 1)), cac)
            kk = jnp.where(kk == m, F32MAX, kk)
        st = W
        while st >= 1:
            pk = jnp.where((lane_w2 & st) == 0, _rot(acc, st), _rot(acc, 2 * W - st))
            pc = jnp.where((lane_w2 & st) == 0, _rot(cac, st), _rot(cac, 2 * W - st))
            lower = (lane_w2 & st) == 0
            take = (lower & (pk < acc)) | (~lower & (pk > acc))
            acc = jnp.where(take, pk, acc)
            cac = jnp.where(take, pc, cac)
            st //= 2
        return acc[:, :W], cac[:, :W]

    _, idxs = jax.lax.fori_loop(jlo_ref[t], jhi_ref[t], tile_body, (v0, i0))
    idx_ref[...] = jnp.minimum(idxs[:, :K], N - 1)


def _topk(xp_bf, sq, sq3, jlo, jhi, row_start, row_end, base_t, nrows,
          interpret=False):
    return pl.pallas_call(
        functools.partial(_topk_body, base_t),
        grid=(nrows // R,),
        in_specs=[
            pl.BlockSpec(memory_space=pltpu.SMEM),
            pl.BlockSpec(memory_space=pltpu.SMEM),
            pl.BlockSpec((R, C), lambda r: (r + base_t, 0)),
            pl.BlockSpec((NPAD, C), lambda r: (0, 0)),
            pl.BlockSpec((R, 1), lambda r: (r + base_t, 0)),
            pl.BlockSpec((NCT, 1, CT), lambda r: (0, 0, 0)),
            pl.BlockSpec((R, 1), lambda r: (r + base_t, 0)),
            pl.BlockSpec((R, 1), lambda r: (r + base_t, 0)),
        ],
        out_specs=pl.BlockSpec((R, K), lambda r: (r, 0)),
        out_shape=jax.ShapeDtypeStruct((nrows, K), jnp.int32),
        interpret=interpret,
    )(jlo, jhi, xp_bf[:N], xp_bf, sq, sq3, row_start, row_end)


def _gather_sc(x, idx_flat_padded, npad):
    mesh = plsc.VectorSubcoreMesh(core_axis_name="c", subcore_axis_name="s")

    @pl.kernel(
        out_type=jax.ShapeDtypeStruct((npad, C), jnp.float32),
        mesh=mesh,
    )
    def kern(x_hbm, i_hbm, o_hbm):
        def body(i_vmem, o_vmem):
            pltpu.sync_copy(x_hbm.at[i_vmem.at[0]], o_vmem)

        pltpu.emit_pipeline(
            body,
            grid=(npad // GWIN,),
            in_specs=[pl.BlockSpec((1, GWIN), index_map=lambda i: (0, i))],
            out_specs=[pl.BlockSpec((GWIN, C), index_map=lambda i: (i, 0))],
            core_axis_name=("c", "s"),
            dimension_semantics=(pltpu.PARALLEL,),
        )(i_hbm, o_hbm)

    return kern(x, idx_flat_padded)


def _final_body(x_ref, g_ref, w1_ref, b1_ref, w2_ref, b2_ref, o_ref):
    xi = x_ref[...]
    xi_bf = xi.astype(jnp.bfloat16)
    w1 = w1_ref[...].astype(jnp.bfloat16)
    w2 = w2_ref[...].astype(jnp.bfloat16)
    b1 = b1_ref[...]
    b2 = b2_ref[...]
    acc = None
    for k in range(K):
        dj = (g_ref[k] - xi).astype(jnp.bfloat16)
        e = jnp.concatenate([xi_bf, dj], axis=1)
        h1 = jnp.maximum(
            jnp.dot(e, w1, preferred_element_type=jnp.float32) + b1, 0.0
        )
        z = jnp.dot(
            h1.astype(jnp.bfloat16), w2, preferred_element_type=jnp.float32
        ) + b2
        h2 = jnp.maximum(z, 0.0)
        acc = h2 if acc is None else jnp.maximum(acc, h2)
    o_ref[...] = acc


def _final(x, G, W1, b1, W2, b2, base_t, nrows, interpret=False):
    return pl.pallas_call(
        _final_body,
        grid=(nrows // R,),
        in_specs=[
            pl.BlockSpec((R, C), lambda r: (r + base_t, 0)),
            pl.BlockSpec((K, R, C), lambda r: (0, r, 0)),
            pl.BlockSpec((2 * C, C), lambda r: (0, 0)),
            pl.BlockSpec((1, C), lambda r: (0, 0)),
            pl.BlockSpec((C, C), lambda r: (0, 0)),
            pl.BlockSpec((1, C), lambda r: (0, 0)),
        ],
        out_specs=pl.BlockSpec((R, C), lambda r: (r, 0)),
        out_shape=jax.ShapeDtypeStruct((nrows, C), jnp.float32),
        interpret=interpret,
    )(x, G, W1, b1.reshape(1, C), W2, b2.reshape(1, C))


def _segment_scalars(batch):
    gids = jnp.arange(8, dtype=batch.dtype)
    seg_start = jnp.searchsorted(batch, gids, side="left").astype(jnp.int32)
    seg_end = jnp.searchsorted(batch, gids, side="right").astype(jnp.int32)
    row_start = seg_start[batch].reshape(N, 1)
    row_end = seg_end[batch].reshape(N, 1)
    first_b = batch[::R]
    last_b = batch[R - 1 :: R]
    jlo = (seg_start[first_b] // CT).astype(jnp.int32)
    jhi = ((seg_end[last_b] + CT - 1) // CT).astype(jnp.int32)
    return jlo, jhi, row_start, row_end


def _gather_pad(idx_half):
    nk = idx_half.shape[0] * K
    npad = -(-nk // GWIN) * GWIN
    idx_km = jnp.transpose(idx_half).reshape(-1)
    return jnp.pad(idx_km, (0, npad - nk)).reshape(1, npad), npad


@jax.jit
def kernel(x, batch, W1, b1, W2, b2):
    batch = batch.astype(jnp.int32)
    xp_bf = jnp.pad(x.astype(jnp.bfloat16), ((0, NPAD - N), (0, 0)))
    jlo, jhi, row_start, row_end = _segment_scalars(batch)

    sq = _sqnorm(x)
    sq3 = jnp.pad(sq.reshape(N), (0, NPAD - N)).reshape(NCT, 1, CT)

    outs = []
    base = 0
    for nh in CHUNKS:
        bt = base // R
        idx = _topk(xp_bf, sq, sq3, jlo, jhi, row_start, row_end, bt, nh)
        isc, npd = _gather_pad(idx)
        G = _gather_sc(x, isc, npd)[: nh * K].reshape(K, nh, C)
        outs.append(_final(x, G, W1, b1, W2, b2, bt, nh))
        base += nh
    return jnp.concatenate(outs, axis=0)


# speedup vs baseline: 1.2490x; 1.0007x over previous
import functools

import jax
import jax.numpy as jnp
from jax.experimental import pallas as pl
from jax.experimental.pallas import tpu as pltpu
from jax.experimental.pallas import tpu_sc as plsc

N = 10000
C = 128
K = 20
R = 400
NT = N // R
CT = 1024
NCT = 10
NPAD = 10240
W = 32
GWIN = 256
F32MAX = float(jnp.finfo(jnp.float32).max)

CHUNKS = (4800, 5200)


def _sqnorm_body(x_ref, sq_ref):
    xr = x_ref[...]
    sq_ref[...] = jnp.sum(xr * xr, axis=1, keepdims=True)


def _sqnorm(x, interpret=False):
    return pl.pallas_call(
        _sqnorm_body,
        grid=(NT,),
        in_specs=[pl.BlockSpec((R, C), lambda r: (r, 0))],
        out_specs=pl.BlockSpec((R, 1), lambda r: (r, 0)),
        out_shape=jax.ShapeDtypeStruct((N, 1), jnp.float32),
        interpret=interpret,
    )(x)


def _rot(x, s):
    return jnp.concatenate([x[:, s:], x[:, :s]], axis=1)


def _topk_body(
    base_t, jlo_ref, jhi_ref, xr_ref, xp_ref, sqr_ref, sqc_ref, rs_ref,
    re_ref, idx_ref
):
    t = pl.program_id(0) + base_t
    xr = xr_ref[...]
    sqr = sqr_ref[...]
    rs = rs_ref[...]
    re = re_ref[...]
    row_ids = t * R + jax.lax.broadcasted_iota(jnp.int32, (R, 1), 0)
    lane_w2 = jax.lax.broadcasted_iota(jnp.int32, (1, 2 * W), 1)

    v0 = jnp.full((R, W), F32MAX, dtype=jnp.float32)
    i0 = jnp.broadcast_to(row_ids, (R, W))

    def tile_body(j, carry):
        v, idxs = carry
        cs = j * CT
        xc = xp_ref[pl.ds(cs, CT), :]
        dots = jax.lax.dot_general(
            xr, xc, (((1,), (1,)), ((), ())),
            preferred_element_type=jnp.float32,
        )
        sqc = sqc_ref[j]
        lane_c = jax.lax.broadcasted_iota(jnp.int32, (1, CT), 1)
        colid = cs + lane_c
        valid = (colid >= rs) & (colid < re)
        s = ((sqr + sqc) - 2.0 * dots + 48.0) * 512.0
        s = jnp.clip(s, 16384.0, 2088000.0)
        ki = s.astype(jnp.int32)
        kf = jax.lax.bitcast_convert_type((ki << 10) | lane_c, jnp.float32)
        kk = jnp.where(valid, kf, F32MAX)

        acc = jnp.concatenate(
            [v, jnp.full((R, W), F32MAX, dtype=jnp.float32)], axis=1
        )
        cac = jnp.concatenate([idxs, idxs], axis=1)
        for q in range(K):
            m = jnp.min(kk, axis=1, keepdims=True)
            tgt = lane_w2 == (2 * W - 1 - q)
            acc = jnp.where(tgt, m, acc)
            mi = jax.lax.bitcast_convert_type(m, jnp.int32)
            cac = jnp.where(tgt, cs + (mi & (CT - 1)), cac)
            kk = jnp.where(kk == m, F32MAX, kk)
        st = W
        while st >= 1:
            pk = jnp.where((lane_w2 & st) == 0, _rot(acc, st), _rot(acc, 2 * W - st))
            pc = jnp.where((lane_w2 & st) == 0, _rot(cac, st), _rot(cac, 2 * W - st))
            lower = (lane_w2 & st) == 0
            take = (lower & (pk < acc)) | (~lower & (pk > acc))
            acc = jnp.where(take, pk, acc)
            cac = jnp.where(take, pc, cac)
            st //= 2
        return acc[:, :W], cac[:, :W]

    _, idxs = jax.lax.fori_loop(jlo_ref[t], jhi_ref[t], tile_body, (v0, i0))
    idx_ref[...] = jnp.minimum(idxs[:, :K], N - 1)


def _topk(xp_bf, sq, sq3, jlo, jhi, row_start, row_end, base_t, nrows,
          interpret=False):
    return pl.pallas_call(
        functools.partial(_topk_body, base_t),
        grid=(nrows // R,),
        in_specs=[
            pl.BlockSpec(memory_space=pltpu.SMEM),
            pl.BlockSpec(memory_space=pltpu.SMEM),
            pl.BlockSpec((R, C), lambda r: (r + base_t, 0)),
            pl.BlockSpec((NPAD, C), lambda r: (0, 0)),
            pl.BlockSpec((R, 1), lambda r: (r + base_t, 0)),
            pl.BlockSpec((NCT, 1, CT), lambda r: (0, 0, 0)),
            pl.BlockSpec((R, 1), lambda r: (r + base_t, 0)),
            pl.BlockSpec((R, 1), lambda r: (r + base_t, 0)),
        ],
        out_specs=pl.BlockSpec((R, K), lambda r: (r, 0)),
        out_shape=jax.ShapeDtypeStruct((nrows, K), jnp.int32),
        interpret=interpret,
    )(jlo, jhi, xp_bf[:N], xp_bf, sq, sq3, row_start, row_end)


def _gather_sc(x, idx_flat_padded, npad):
    mesh = plsc.VectorSubcoreMesh(core_axis_name="c", subcore_axis_name="s")

    @pl.kernel(
        out_type=jax.ShapeDtypeStruct((npad, C), jnp.float32),
        mesh=mesh,
    )
    def kern(x_hbm, i_hbm, o_hbm):
        def body(i_vmem, o_vmem):
            pltpu.sync_copy(x_hbm.at[i_vmem.at[0]], o_vmem)

        pltpu.emit_pipeline(
            body,
            grid=(npad // GWIN,),
            in_specs=[pl.BlockSpec((1, GWIN), index_map=lambda i: (0, i))],
            out_specs=[pl.BlockSpec((GWIN, C), index_map=lambda i: (i, 0))],
            core_axis_name=("c", "s"),
            dimension_semantics=(pltpu.PARALLEL,),
        )(i_hbm, o_hbm)

    return kern(x, idx_flat_padded)


def _final_body(x_ref, g_ref, w1_ref, b1_ref, w2_ref, b2_ref, o_ref):
    xi = x_ref[...]
    xi_bf = xi.astype(jnp.bfloat16)
    w1 = w1_ref[...].astype(jnp.bfloat16)
    w2 = w2_ref[...].astype(jnp.bfloat16)
    b1 = b1_ref[...]
    b2 = b2_ref[...]
    acc = None
    for k in range(K):
        dj = (g_ref[k] - xi).astype(jnp.bfloat16)
        e = jnp.concatenate([xi_bf, dj], axis=1)
        h1 = jnp.maximum(
            jnp.dot(e, w1, preferred_element_type=jnp.float32) + b1, 0.0
        )
        z = jnp.dot(
            h1.astype(jnp.bfloat16), w2, preferred_element_type=jnp.float32
        ) + b2
        h2 = jnp.maximum(z, 0.0)
        acc = h2 if acc is None else jnp.maximum(acc, h2)
    o_ref[...] = acc


def _final(x, G, W1, b1, W2, b2, base_t, nrows, interpret=False):
    return pl.pallas_call(
        _final_body,
        grid=(nrows // R,),
        in_specs=[
            pl.BlockSpec((R, C), lambda r: (r + base_t, 0)),
            pl.BlockSpec((K, R, C), lambda r: (0, r, 0)),
            pl.BlockSpec((2 * C, C), lambda r: (0, 0)),
            pl.BlockSpec((1, C), lambda r: (0, 0)),
            pl.BlockSpec((C, C), lambda r: (0, 0)),
            pl.BlockSpec((1, C), lambda r: (0, 0)),
        ],
        out_specs=pl.BlockSpec((R, C), lambda r: (r, 0)),
        out_shape=jax.ShapeDtypeStruct((nrows, C), jnp.float32),
        interpret=interpret,
    )(x, G, W1, b1.reshape(1, C), W2, b2.reshape(1, C))


def _segment_scalars(batch):
    gids = jnp.arange(8, dtype=batch.dtype)
    seg_start = jnp.searchsorted(batch, gids, side="left").astype(jnp.int32)
    seg_end = jnp.searchsorted(batch, gids, side="right").astype(jnp.int32)
    row_start = seg_start[batch].reshape(N, 1)
    row_end = seg_end[batch].reshape(N, 1)
    first_b = batch[::R]
    last_b = batch[R - 1 :: R]
    jlo = (seg_start[first_b] // CT).astype(jnp.int32)
    jhi = ((seg_end[last_b] + CT - 1) // CT).astype(jnp.int32)
    return jlo, jhi, row_start, row_end


def _gather_pad(idx_half):
    nk = idx_half.shape[0] * K
    npad = -(-nk // GWIN) * GWIN
    idx_km = jnp.transpose(idx_half).reshape(-1)
    return jnp.pad(idx_km, (0, npad - nk)).reshape(1, npad), npad


@jax.jit
def kernel(x, batch, W1, b1, W2, b2):
    batch = batch.astype(jnp.int32)
    xp_bf = jnp.pad(x.astype(jnp.bfloat16), ((0, NPAD - N), (0, 0)))
    jlo, jhi, row_start, row_end = _segment_scalars(batch)

    sq = _sqnorm(x)
    sq3 = jnp.pad(sq.reshape(N), (0, NPAD - N)).reshape(NCT, 1, CT)

    outs = []
    base = 0
    for nh in CHUNKS:
        bt = base // R
        idx = _topk(xp_bf, sq, sq3, jlo, jhi, row_start, row_end, bt, nh)
        isc, npd = _gather_pad(idx)
        G = _gather_sc(x, isc, npd)[: nh * K].reshape(K, nh, C)
        outs.append(_final(x, G, W1, b1, W2, b2, bt, nh))
        base += nh
    return jnp.concatenate(outs, axis=0)
